# initial kernel scaffold (unmeasured)
import jax
import jax.numpy as jnp
from jax import lax
from jax.experimental import pallas as pl
from jax.experimental.pallas import tpu as pltpu


def kernel(
    t,
):
    def body(*refs):
        pass

    out_shape = jax.ShapeDtypeStruct(..., jnp.float32)
    return pl.pallas_call(body, out_shape=out_shape)(...)


# baseline (device time: 153799 ns/iter reference)
import jax
import jax.numpy as jnp
from jax import lax
from jax.experimental import pallas as pl
from jax.experimental.pallas import tpu as pltpu

N_DEV = 4


def kernel(t):
    m, n = t.shape

    def body(x_ref, out_ref, comm_ref, send_sems, recv_sems):
        my = lax.axis_index("i")
        left = (my - 1) % N_DEV
        right = (my + 1) % N_DEV

        barrier_sem = pltpu.get_barrier_semaphore()
        for nbr in [left, right]:
            pl.semaphore_signal(
                barrier_sem, inc=1,
                device_id=(nbr,), device_id_type=pl.DeviceIdType.MESH,
            )
        pl.semaphore_wait(barrier_sem, 2)

        comm_ref[0, :, :] = x_ref[:, :].astype(jnp.bfloat16)

        for h in range(N_DEV - 1):
            rdma = pltpu.make_async_remote_copy(
                src_ref=comm_ref.at[h],
                dst_ref=comm_ref.at[h + 1],
                send_sem=send_sems.at[h],
                recv_sem=recv_sems.at[h + 1],
                device_id=(right,),
                device_id_type=pl.DeviceIdType.MESH,
            )
            rdma.start()
            rdma.wait()

        s = x_ref[:, :]
        for h in range(N_DEV - 1):
            s = s + comm_ref[h + 1, :, :].astype(jnp.float32)
        r = jnp.maximum(s, 0.0)
        out_ref[:, :] = jnp.tanh(s) * s * s + r * r * r

    return pl.pallas_call(
        body,
        out_shape=jax.ShapeDtypeStruct((m, n), jnp.float32),
        in_specs=[pl.BlockSpec(memory_space=pltpu.VMEM)],
        out_specs=pl.BlockSpec(memory_space=pltpu.VMEM),
        scratch_shapes=[
            pltpu.VMEM((N_DEV, m, n), jnp.bfloat16),
            pltpu.SemaphoreType.DMA((N_DEV,)),
            pltpu.SemaphoreType.DMA((N_DEV,)),
        ],
        compiler_params=pltpu.CompilerParams(collective_id=0),
    )(t)


# device time: 52047 ns/iter; 2.9550x vs baseline; 2.9550x over previous
import jax
import jax.numpy as jnp
from jax import lax
from jax.experimental import pallas as pl
from jax.experimental.pallas import tpu as pltpu

N_DEV = 4


def kernel(t):
    m, n = t.shape
    mh = m // 2
    mq = m // 4
    me = m // 8

    def body(x_ref, out_ref, xbf, r1a, r1b, ha, hb, r2a, r2b, gbuf,
             ssem, rsem):
        my = lax.axis_index("i")
        p1 = my ^ 1
        p2 = 3 - my
        b1 = (my ^ (my // 2)) % 2
        b2 = (my // 2) % 2

        barrier_sem = pltpu.get_barrier_semaphore()
        for nbr in [p1, p2]:
            pl.semaphore_signal(
                barrier_sem, inc=1,
                device_id=(nbr,), device_id_type=pl.DeviceIdType.MESH,
            )
        pl.semaphore_wait(barrier_sem, 2)

        def exch(idx, src, dst, partner):
            rdma = pltpu.make_async_remote_copy(
                src_ref=src, dst_ref=dst,
                send_sem=ssem.at[idx], recv_sem=rsem.at[idx],
                device_id=(partner,), device_id_type=pl.DeviceIdType.MESH,
            )
            rdma.start()
            return rdma

        xbf[:, :] = x_ref[:, :].astype(jnp.bfloat16)

        da = exch(0, xbf.at[pl.ds((1 - b1) * mq, mq)], r1a, p1)
        db = exch(1, xbf.at[pl.ds(mh + (1 - b2) * mq, mq)], r1b, p2)
        da.wait()
        db.wait()

        ha[:, :] = (x_ref[pl.ds(b1 * mq, mq), :]
                    + r1a[:, :].astype(jnp.float32)).astype(jnp.bfloat16)
        hb[:, :] = (x_ref[pl.ds(mh + b2 * mq, mq), :]
                    + r1b[:, :].astype(jnp.float32)).astype(jnp.bfloat16)

        da = exch(2, ha.at[pl.ds((1 - b2) * me, me)], r2a, p2)
        db = exch(3, hb.at[pl.ds((1 - b1) * me, me)], r2b, p1)
        da.wait()
        db.wait()

        qa = (ha[pl.ds(b2 * me, me), :].astype(jnp.float32)
              + r2a[:, :].astype(jnp.float32))
        qb = (hb[pl.ds(b1 * me, me), :].astype(jnp.float32)
              + r2b[:, :].astype(jnp.float32))

        def f(s):
            r = jnp.maximum(s, 0.0)
            return jnp.tanh(s) * s * s + r * r * r

        qa_start = b1 * mq + b2 * me
        qb_start = mh + b2 * mq + b1 * me
        gbuf[pl.ds(qa_start, me), :] = f(qa).astype(jnp.bfloat16)
        gbuf[pl.ds(qb_start, me), :] = f(qb).astype(jnp.bfloat16)

        da = exch(4, gbuf.at[pl.ds(qa_start, me)],
                  gbuf.at[pl.ds(qa_start, me)], p2)
        db = exch(5, gbuf.at[pl.ds(qb_start, me)],
                  gbuf.at[pl.ds(qb_start, me)], p1)
        da.wait()
        db.wait()

        da = exch(6, gbuf.at[pl.ds(b1 * mq, mq)],
                  gbuf.at[pl.ds(b1 * mq, mq)], p1)
        db = exch(7, gbuf.at[pl.ds(mh + b2 * mq, mq)],
                  gbuf.at[pl.ds(mh + b2 * mq, mq)], p2)
        da.wait()
        db.wait()

        out_ref[:, :] = gbuf[:, :].astype(jnp.float32)

    return pl.pallas_call(
        body,
        out_shape=jax.ShapeDtypeStruct((m, n), jnp.float32),
        in_specs=[pl.BlockSpec(memory_space=pltpu.VMEM)],
        out_specs=pl.BlockSpec(memory_space=pltpu.VMEM),
        scratch_shapes=[
            pltpu.VMEM((m, n), jnp.bfloat16),
            pltpu.VMEM((mq, n), jnp.bfloat16),
            pltpu.VMEM((mq, n), jnp.bfloat16),
            pltpu.VMEM((mq, n), jnp.bfloat16),
            pltpu.VMEM((mq, n), jnp.bfloat16),
            pltpu.VMEM((me, n), jnp.bfloat16),
            pltpu.VMEM((me, n), jnp.bfloat16),
            pltpu.VMEM((m, n), jnp.bfloat16),
            pltpu.SemaphoreType.DMA((8,)),
            pltpu.SemaphoreType.DMA((8,)),
        ],
        compiler_params=pltpu.CompilerParams(collective_id=0),
    )(t)


# device time: 49638 ns/iter; 3.0984x vs baseline; 1.0485x over previous
import jax
import jax.numpy as jnp
from jax import lax
from jax.experimental import pallas as pl
from jax.experimental.pallas import tpu as pltpu

N_DEV = 4


def kernel(t):
    m, n = t.shape
    mh = m // 2
    mq = m // 4
    me = m // 8

    def body(x_ref, out_ref, s1a, s1b, r1a, r1b, s2a, s2b, r2a, r2b,
             ssem, rsem):
        my = lax.axis_index("i")
        p1 = my ^ 1
        p2 = 3 - my
        b1 = (my ^ (my // 2)) % 2
        b2 = (my // 2) % 2

        barrier_sem = pltpu.get_barrier_semaphore()
        for nbr in [p1, p2]:
            pl.semaphore_signal(
                barrier_sem, inc=1,
                device_id=(nbr,), device_id_type=pl.DeviceIdType.MESH,
            )
        pl.semaphore_wait(barrier_sem, 2)

        def exch(idx, src, dst, partner):
            rdma = pltpu.make_async_remote_copy(
                src_ref=src, dst_ref=dst,
                send_sem=ssem.at[idx], recv_sem=rsem.at[idx],
                device_id=(partner,), device_id_type=pl.DeviceIdType.MESH,
            )
            rdma.start()
            return rdma

        f32 = jnp.float32
        bf16 = jnp.bfloat16

        s1a[:, :] = x_ref[pl.ds((1 - b1) * mq, mq), :].astype(bf16)
        da = exch(0, s1a, r1a, p1)
        s1b[:, :] = x_ref[pl.ds(mh + (1 - b2) * mq, mq), :].astype(bf16)
        db = exch(1, s1b, r1b, p2)
        da.wait()
        db.wait()

        s2a[:, :] = (x_ref[pl.ds(b1 * mq + (1 - b2) * me, me), :]
                     + r1a[pl.ds((1 - b2) * me, me), :].astype(f32)
                     ).astype(bf16)
        da = exch(2, s2a, r2a, p2)
        s2b[:, :] = (x_ref[pl.ds(mh + b2 * mq + (1 - b1) * me, me), :]
                     + r1b[pl.ds((1 - b1) * me, me), :].astype(f32)
                     ).astype(bf16)
        db = exch(3, s2b, r2b, p1)

        pa = (x_ref[pl.ds(b1 * mq + b2 * me, me), :]
              + r1a[pl.ds(b2 * me, me), :].astype(f32))
        pb = (x_ref[pl.ds(mh + b2 * mq + b1 * me, me), :]
              + r1b[pl.ds(b1 * me, me), :].astype(f32))

        da.wait()
        db.wait()

        def f(s):
            r = jnp.maximum(s, 0.0)
            return jnp.tanh(s) * s * s + r * r * r

        qa_start = b1 * mq + b2 * me
        qb_start = mh + b2 * mq + b1 * me
        out_ref[pl.ds(qa_start, me), :] = f(pa + r2a[:, :].astype(f32)
                                            ).astype(bf16)
        out_ref[pl.ds(qb_start, me), :] = f(pb + r2b[:, :].astype(f32)
                                            ).astype(bf16)

        da = exch(4, out_ref.at[pl.ds(qa_start, me)],
                  out_ref.at[pl.ds(qa_start, me)], p2)
        db = exch(5, out_ref.at[pl.ds(qb_start, me)],
                  out_ref.at[pl.ds(qb_start, me)], p1)
        da.wait()
        db.wait()

        da = exch(6, out_ref.at[pl.ds(b1 * mq, mq)],
                  out_ref.at[pl.ds(b1 * mq, mq)], p1)
        db = exch(7, out_ref.at[pl.ds(mh + b2 * mq, mq)],
                  out_ref.at[pl.ds(mh + b2 * mq, mq)], p2)
        da.wait()
        db.wait()

    return pl.pallas_call(
        body,
        out_shape=jax.ShapeDtypeStruct((m, n), jnp.bfloat16),
        in_specs=[pl.BlockSpec(memory_space=pltpu.VMEM)],
        out_specs=pl.BlockSpec(memory_space=pltpu.VMEM),
        scratch_shapes=[
            pltpu.VMEM((mq, n), jnp.bfloat16),
            pltpu.VMEM((mq, n), jnp.bfloat16),
            pltpu.VMEM((mq, n), jnp.bfloat16),
            pltpu.VMEM((mq, n), jnp.bfloat16),
            pltpu.VMEM((me, n), jnp.bfloat16),
            pltpu.VMEM((me, n), jnp.bfloat16),
            pltpu.VMEM((me, n), jnp.bfloat16),
            pltpu.VMEM((me, n), jnp.bfloat16),
            pltpu.SemaphoreType.DMA((8,)),
            pltpu.SemaphoreType.DMA((8,)),
        ],
        compiler_params=pltpu.CompilerParams(collective_id=0),
    )(t)


# device time: 44264 ns/iter; 3.4746x vs baseline; 1.1214x over previous
import jax
import jax.numpy as jnp
from jax import lax
from jax.experimental import pallas as pl
from jax.experimental.pallas import tpu as pltpu

N_DEV = 4
N_CHUNKS = 2


def kernel(t):
    m, n = t.shape
    mh = m // 2
    mq = m // 4
    me = m // 8
    nc = n // N_CHUNKS

    def body(x_ref, out_ref, s1a, s1b, r1a, r1b, s2a, s2b, r2a, r2b,
             ssem, rsem):
        my = lax.axis_index("i")
        p1 = my ^ 1
        p2 = 3 - my
        b1 = (my ^ (my // 2)) % 2
        b2 = (my // 2) % 2

        f32 = jnp.float32
        bf16 = jnp.bfloat16
        qa_start = b1 * mq + b2 * me
        qb_start = mh + b2 * mq + b1 * me

        barrier_sem = pltpu.get_barrier_semaphore()
        for nbr in [p1, p2]:
            pl.semaphore_signal(
                barrier_sem, inc=1,
                device_id=(nbr,), device_id_type=pl.DeviceIdType.MESH,
            )
        pl.semaphore_wait(barrier_sem, 2)

        def exch(idx, src, dst, partner):
            rdma = pltpu.make_async_remote_copy(
                src_ref=src, dst_ref=dst,
                send_sem=ssem.at[idx], recv_sem=rsem.at[idx],
                device_id=(partner,), device_id_type=pl.DeviceIdType.MESH,
            )
            rdma.start()
            return rdma

        def cs(c):
            return pl.ds(c * nc, nc)


        def rs1_issue(c):
            s1a[:, cs(c)] = x_ref[pl.ds((1 - b1) * mq, mq), cs(c)
                                  ].astype(bf16)
            da = exch(0 + c, s1a.at[:, cs(c)], r1a.at[:, cs(c)], p1)
            s1b[:, cs(c)] = x_ref[pl.ds(mh + (1 - b2) * mq, mq), cs(c)
                                  ].astype(bf16)
            db = exch(2 + c, s1b.at[:, cs(c)], r1b.at[:, cs(c)], p2)
            return da, db

        def rs2_issue(c):
            s2a[:, cs(c)] = (
                x_ref[pl.ds(b1 * mq + (1 - b2) * me, me), cs(c)]
                + r1a[pl.ds((1 - b2) * me, me), cs(c)].astype(f32)
            ).astype(bf16)
            da = exch(4 + c, s2a.at[:, cs(c)], r2a.at[:, cs(c)], p2)
            s2b[:, cs(c)] = (
                x_ref[pl.ds(mh + b2 * mq + (1 - b1) * me, me), cs(c)]
                + r1b[pl.ds((1 - b1) * me, me), cs(c)].astype(f32)
            ).astype(bf16)
            db = exch(6 + c, s2b.at[:, cs(c)], r2b.at[:, cs(c)], p1)
            return da, db

        def f(s):
            r = jnp.maximum(s, 0.0)
            return jnp.tanh(s) * s * s + r * r * r

        def ag1_issue(c):
            out_ref[pl.ds(qa_start, me), cs(c)] = f(
                x_ref[pl.ds(b1 * mq + b2 * me, me), cs(c)]
                + r1a[pl.ds(b2 * me, me), cs(c)].astype(f32)
                + r2a[:, cs(c)].astype(f32)
            ).astype(bf16)
            da = exch(8 + c, out_ref.at[pl.ds(qa_start, me), cs(c)],
                      out_ref.at[pl.ds(qa_start, me), cs(c)], p2)
            out_ref[pl.ds(qb_start, me), cs(c)] = f(
                x_ref[pl.ds(mh + b2 * mq + b1 * me, me), cs(c)]
                + r1b[pl.ds(b1 * me, me), cs(c)].astype(f32)
                + r2b[:, cs(c)].astype(f32)
            ).astype(bf16)
            db = exch(10 + c, out_ref.at[pl.ds(qb_start, me), cs(c)],
                      out_ref.at[pl.ds(qb_start, me), cs(c)], p1)
            return da, db

        def ag2_issue(c):
            da = exch(12 + c, out_ref.at[pl.ds(b1 * mq, mq), cs(c)],
                      out_ref.at[pl.ds(b1 * mq, mq), cs(c)], p1)
            db = exch(14 + c, out_ref.at[pl.ds(mh + b2 * mq, mq), cs(c)],
                      out_ref.at[pl.ds(mh + b2 * mq, mq), cs(c)], p2)
            return da, db

        def wait(pair):
            pair[0].wait()
            pair[1].wait()

        rs1_0 = rs1_issue(0)
        rs1_1 = rs1_issue(1)
        wait(rs1_0)
        rs2_0 = rs2_issue(0)
        wait(rs1_1)
        rs2_1 = rs2_issue(1)
        wait(rs2_0)
        ag1_0 = ag1_issue(0)
        wait(rs2_1)
        ag1_1 = ag1_issue(1)
        wait(ag1_0)
        ag2_0 = ag2_issue(0)
        wait(ag1_1)
        ag2_1 = ag2_issue(1)
        wait(ag2_0)
        wait(ag2_1)

    return pl.pallas_call(
        body,
        out_shape=jax.ShapeDtypeStruct((m, n), jnp.bfloat16),
        in_specs=[pl.BlockSpec(memory_space=pltpu.VMEM)],
        out_specs=pl.BlockSpec(memory_space=pltpu.VMEM),
        scratch_shapes=[
            pltpu.VMEM((mq, n), jnp.bfloat16),
            pltpu.VMEM((mq, n), jnp.bfloat16),
            pltpu.VMEM((mq, n), jnp.bfloat16),
            pltpu.VMEM((mq, n), jnp.bfloat16),
            pltpu.VMEM((me, n), jnp.bfloat16),
            pltpu.VMEM((me, n), jnp.bfloat16),
            pltpu.VMEM((me, n), jnp.bfloat16),
            pltpu.VMEM((me, n), jnp.bfloat16),
            pltpu.SemaphoreType.DMA((16,)),
            pltpu.SemaphoreType.DMA((16,)),
        ],
        compiler_params=pltpu.CompilerParams(collective_id=0),
    )(t)
